# baseline (speedup 1.0000x reference)
import functools

import jax
import jax.numpy as jnp
import numpy as np
from jax.experimental import pallas as pl

_POOLING_COUNTS = (15, 8, 8)
_N = 64


def _mask2d_np(N, pooling_counts):
    m = np.zeros((N, N), dtype=bool)
    m[np.arange(N), np.arange(N)] = True
    stride, offset = 1, 0
    for c in pooling_counts:
        for _ in range(c):
            offset += stride
            i = np.arange(0, N - offset, stride)
            m[i, i + offset] = True
        stride *= 2
    return m


_MASK = _mask2d_np(_N, _POOLING_COUNTS)


def _map_body(x_ref, mask_ref, out_ref):
    xb = x_ref[0]
    bd = xb.shape[0]
    N = _N
    lane = jax.lax.broadcasted_iota(jnp.int32, (bd, N), 1)
    R = jnp.full((bd, N), -jnp.inf, dtype=xb.dtype)
    for i in range(N - 1, -1, -1):
        xi = xb[:, i][:, None]
        S = jnp.where(lane >= i, xi, -jnp.inf)
        R = jnp.maximum(R, S)
        mrow = mask_ref[i, :][None, :]
        out_ref[0, :, i, :] = jnp.where(mrow > 0, R, 0.0)


@functools.partial(jax.jit, static_argnames=())
def kernel(x):
    B, D, N = x.shape
    bd = 32
    grid = (B, D // bd)
    mask_f32 = jnp.asarray(_MASK, dtype=x.dtype)
    map2d = pl.pallas_call(
        _map_body,
        grid=grid,
        in_specs=[
            pl.BlockSpec((1, bd, N), lambda b, d: (b, d, 0)),
            pl.BlockSpec((N, N), lambda b, d: (0, 0)),
        ],
        out_specs=pl.BlockSpec((1, bd, N, N), lambda b, d: (b, d, 0, 0)),
        out_shape=jax.ShapeDtypeStruct((B, D, N, N), x.dtype),
    )(x, mask_f32)
    mask2d = jnp.broadcast_to(jnp.asarray(_MASK)[None, None, :, :], (B, 1, N, N))
    return (map2d, mask2d)
